# Initial kernel scaffold; baseline (speedup 1.0000x reference)
import jax
import jax.numpy as jnp
from jax.experimental import pallas as pl


def kernel(inputs, embeddings, idf):
    raise NotImplementedError("write your pallas kernel here")


# speedup vs baseline: 14.6185x; 14.6185x over previous
import functools

import jax
import jax.numpy as jnp
from jax import lax
from jax.experimental import pallas as pl
from jax.experimental.pallas import tpu as pltpu
from jax.experimental.pallas import tpu_sc as plsc

_NC = 2
_NS = 16
_NW = _NC * _NS

_IDXM = 128


def _scale_body(emb_ref, idf_ref, out_ref):
    out_ref[...] = emb_ref[...] * idf_ref[...]


@functools.lru_cache(maxsize=None)
def _make_gather(V, D, N):
    npw = N // _NW
    idx_rows = npw // _IDXM
    chunk = 512
    gpc = chunk // _IDXM
    nchunk = npw // chunk

    mesh = plsc.VectorSubcoreMesh(core_axis_name="c", subcore_axis_name="s")

    @functools.partial(
        pl.kernel,
        out_type=jax.ShapeDtypeStruct((N, D), jnp.float32),
        mesh=mesh,
        compiler_params=pltpu.CompilerParams(use_tc_tiling_on_sc=False),
        scratch_types=[
            pltpu.VMEM((idx_rows, _IDXM), jnp.int32),
            pltpu.VMEM((chunk, D), jnp.float32),
            pltpu.VMEM((chunk, D), jnp.float32),
            pltpu.SemaphoreType.DMA,
            pltpu.SemaphoreType.DMA,
        ],
    )
    def gather(table_hbm, idx_hbm, out_hbm, idx_v, buf_a, buf_b, sem_a, sem_b):
        wid = lax.axis_index("s") * _NC + lax.axis_index("c")
        base = wid * npw

        pltpu.sync_copy(idx_hbm.at[pl.ds(wid * idx_rows, idx_rows)], idx_v)

        def fire(c, buf, sem):
            for j in range(gpc):
                pltpu.async_copy(
                    table_hbm.at[idx_v.at[c * gpc + j]],
                    buf.at[pl.ds(j * _IDXM, _IDXM)],
                    sem,
                )

        def drain(buf, sem):
            pltpu.make_async_copy(out_hbm.at[pl.ds(0, chunk)], buf, sem).wait()

        fire(0, buf_a, sem_a)

        def body(k, carry):
            i = 2 * k
            fire(i + 1, buf_b, sem_b)
            drain(buf_a, sem_a)
            pltpu.sync_copy(buf_a, out_hbm.at[pl.ds(base + i * chunk, chunk)])

            @pl.when(i < nchunk - 2)
            def _():
                fire(i + 2, buf_a, sem_a)

            drain(buf_b, sem_b)
            pltpu.sync_copy(buf_b, out_hbm.at[pl.ds(base + (i + 1) * chunk, chunk)])
            return carry

        lax.fori_loop(0, nchunk // 2, body, 0)

    return gather


def kernel(inputs, embeddings, idf):
    B, H = inputs.shape
    V, D = embeddings.shape
    N = B * H

    scaled = pl.pallas_call(
        _scale_body,
        out_shape=jax.ShapeDtypeStruct((V, D), jnp.float32),
    )(embeddings, idf)

    idx2d = inputs.reshape(N // _IDXM, _IDXM)
    out = _make_gather(V, D, N)(scaled, idx2d)
    return out.reshape(B, H, D)


# speedup vs baseline: 14.6952x; 1.0052x over previous
import functools

import jax
import jax.numpy as jnp
from jax import lax
from jax.experimental import pallas as pl
from jax.experimental.pallas import tpu as pltpu
from jax.experimental.pallas import tpu_sc as plsc

_NC = 2
_NS = 16
_NW = _NC * _NS

_IDXM = 80
_BCHUNK = 8


def _scale_body(emb_ref, idf_ref, out_ref):
    out_ref[...] = emb_ref[...] * idf_ref[...]


@functools.lru_cache(maxsize=None)
def _make_gather(V, D, B, H):
    bpw = B // _NW
    npw = bpw * H
    idx_rows = npw // _IDXM
    rows = _BCHUNK * H
    gpc = rows // _IDXM
    nchunk = bpw // _BCHUNK

    mesh = plsc.VectorSubcoreMesh(core_axis_name="c", subcore_axis_name="s")

    @functools.partial(
        pl.kernel,
        out_type=jax.ShapeDtypeStruct((B, H, D), jnp.float32),
        mesh=mesh,
        compiler_params=pltpu.CompilerParams(use_tc_tiling_on_sc=False),
        scratch_types=[
            pltpu.VMEM((idx_rows, _IDXM), jnp.int32),
            pltpu.VMEM((rows, D), jnp.float32),
            pltpu.VMEM((rows, D), jnp.float32),
            pltpu.SemaphoreType.DMA,
            pltpu.SemaphoreType.DMA,
            pltpu.SemaphoreType.DMA,
            pltpu.SemaphoreType.DMA,
        ],
    )
    def gather(table_hbm, idx_hbm, out_hbm, idx_v, buf_a, buf_b,
               gsem_a, gsem_b, osem_a, osem_b):
        wid = lax.axis_index("s") * _NC + lax.axis_index("c")
        b_base = wid * bpw

        pltpu.sync_copy(idx_hbm.at[pl.ds(wid * idx_rows, idx_rows)], idx_v)

        def fire_g(c, buf, sem):
            for j in range(gpc):
                pltpu.async_copy(
                    table_hbm.at[idx_v.at[c * gpc + j]],
                    buf.at[pl.ds(j * _IDXM, _IDXM)],
                    sem,
                )

        def drain_g(buf, sem):
            for j in range(gpc):
                pltpu.make_async_copy(
                    table_hbm.at[idx_v.at[0]], buf.at[pl.ds(j * _IDXM, _IDXM)], sem
                ).wait()

        def fire_s(c, buf, sem):
            for i in range(_BCHUNK):
                pltpu.async_copy(
                    buf.at[pl.ds(i * H, H)],
                    out_hbm.at[b_base + c * _BCHUNK + i],
                    sem,
                )

        def drain_s(c, buf, sem):
            for i in range(_BCHUNK):
                pltpu.make_async_copy(
                    buf.at[pl.ds(i * H, H)],
                    out_hbm.at[b_base + c * _BCHUNK + i],
                    sem,
                ).wait()

        fire_g(0, buf_a, gsem_a)
        fire_g(1, buf_b, gsem_b)

        def body(k, carry):
            i = 2 * k
            drain_g(buf_a, gsem_a)
            fire_s(i, buf_a, osem_a)
            drain_g(buf_b, gsem_b)
            fire_s(i + 1, buf_b, osem_b)

            @pl.when(i < nchunk - 2)
            def _():
                drain_s(i, buf_a, osem_a)
                fire_g(i + 2, buf_a, gsem_a)
                drain_s(i + 1, buf_b, osem_b)
                fire_g(i + 3, buf_b, gsem_b)

            return carry

        lax.fori_loop(0, nchunk // 2, body, 0)
        drain_s(nchunk - 2, buf_a, osem_a)
        drain_s(nchunk - 1, buf_b, osem_b)

    return gather


def kernel(inputs, embeddings, idf):
    B, H = inputs.shape
    V, D = embeddings.shape

    scaled = pl.pallas_call(
        _scale_body,
        out_shape=jax.ShapeDtypeStruct((V, D), jnp.float32),
    )(embeddings, idf)

    idx2d = inputs.reshape((B * H) // _IDXM, _IDXM)
    return _make_gather(V, D, B, H)(scaled, idx2d)


# speedup vs baseline: 15.1177x; 1.0288x over previous
import functools

import jax
import jax.numpy as jnp
from jax import lax
from jax.experimental import pallas as pl
from jax.experimental.pallas import tpu as pltpu
from jax.experimental.pallas import tpu_sc as plsc

_NC = 2
_NS = 16
_NW = _NC * _NS

_DP = 128
_BCHUNK = 2


def _scale_pad_body(emb_ref, idf_ref, out_ref):
    scaled = emb_ref[...] * idf_ref[...]
    out_ref[...] = jnp.concatenate([scaled, jnp.zeros_like(scaled)], axis=-1)


@functools.lru_cache(maxsize=None)
def _make_gather(V, D, B, H):
    bpw = B // _NW
    rows = _BCHUNK * H
    nchunk = bpw // _BCHUNK
    lanes = D // 16

    mesh = plsc.VectorSubcoreMesh(core_axis_name="c", subcore_axis_name="s")

    @functools.partial(
        pl.kernel,
        out_type=jax.ShapeDtypeStruct((B, H, D), jnp.float32),
        mesh=mesh,
        compiler_params=pltpu.CompilerParams(use_tc_tiling_on_sc=True),
        scratch_types=[
            pltpu.VMEM((nchunk, rows), jnp.int32),
            pltpu.VMEM((rows, _DP), jnp.float32),
            pltpu.VMEM((rows, _DP), jnp.float32),
            pltpu.VMEM((rows, D), jnp.float32),
            pltpu.VMEM((rows, D), jnp.float32),
            pltpu.SemaphoreType.DMA,
            pltpu.SemaphoreType.DMA,
            pltpu.SemaphoreType.DMA,
            pltpu.SemaphoreType.DMA,
        ],
    )
    def gather(table_hbm, idx_hbm, out_hbm, idx_v, gbuf_a, gbuf_b,
               sbuf_a, sbuf_b, gsem_a, gsem_b, osem_a, osem_b):
        wid = lax.axis_index("s") * _NC + lax.axis_index("c")
        b_base = wid * bpw

        pltpu.sync_copy(idx_hbm.at[pl.ds(wid * nchunk, nchunk)], idx_v)

        def fire_g(c, gbuf, sem):
            pltpu.async_copy(table_hbm.at[idx_v.at[c]], gbuf, sem)

        def drain_g(gbuf, sem):
            pltpu.make_async_copy(table_hbm.at[idx_v.at[0]], gbuf, sem).wait()

        def shuffle(gbuf, sbuf):
            def body(r, carry):
                for c in range(lanes):
                    sbuf[r, pl.ds(c * 16, 16)] = gbuf[r, pl.ds(c * 16, 16)]
                return carry
            lax.fori_loop(0, rows, body, 0, unroll=2)

        def fire_s(c, sbuf, sem):
            for i in range(_BCHUNK):
                pltpu.async_copy(
                    sbuf.at[pl.ds(i * H, H)],
                    out_hbm.at[b_base + c * _BCHUNK + i],
                    sem,
                )

        def drain_s(c, sbuf, sem):
            for i in range(_BCHUNK):
                pltpu.make_async_copy(
                    sbuf.at[pl.ds(i * H, H)],
                    out_hbm.at[b_base + c * _BCHUNK + i],
                    sem,
                ).wait()

        fire_g(0, gbuf_a, gsem_a)
        fire_g(1, gbuf_b, gsem_b)

        def half(k, i, gbuf, sbuf, gsem, osem):
            @pl.when(k > 0)
            def _():
                drain_s(i - 2, sbuf, osem)

            drain_g(gbuf, gsem)
            shuffle(gbuf, sbuf)

            @pl.when(i < nchunk - 2)
            def _():
                fire_g(i + 2, gbuf, gsem)

            fire_s(i, sbuf, osem)

        def body(k, carry):
            i = 2 * k
            half(k, i, gbuf_a, sbuf_a, gsem_a, osem_a)
            half(k, i + 1, gbuf_b, sbuf_b, gsem_b, osem_b)
            return carry

        lax.fori_loop(0, nchunk // 2, body, 0)
        drain_s(nchunk - 2, sbuf_a, osem_a)
        drain_s(nchunk - 1, sbuf_b, osem_b)

    return gather


def kernel(inputs, embeddings, idf):
    B, H = inputs.shape
    V, D = embeddings.shape
    rows = _BCHUNK * H

    scaled = pl.pallas_call(
        _scale_pad_body,
        out_shape=jax.ShapeDtypeStruct((V, _DP), jnp.float32),
    )(embeddings, idf)

    idx2d = inputs.reshape((B * H) // rows, rows)
    return _make_gather(V, D, B, H)(scaled, idx2d)
